# Initial kernel scaffold; baseline (speedup 1.0000x reference)
import jax
import jax.numpy as jnp
from jax.experimental import pallas as pl


def kernel(x_nchw, conv1_w, conv1_b, conv2_w, conv2_b, fc1_w, fc1_b, fc2_w, fc2_b):
    raise NotImplementedError("write your pallas kernel here")


# speedup vs baseline: 1.0304x; 1.0304x over previous
import functools

import jax
import jax.numpy as jnp
from jax.experimental import pallas as pl
from jax.experimental.pallas import tpu as pltpu


def _rup(x, m):
    return (x + m - 1) // m * m


H1, W1 = 28, 28
WP1 = 32
HW1 = H1 * WP1
R1 = _rup((H1 + 4) * WP1 + 4, 8)

H2, W2 = 14, 14
WP2 = 24
OFF2 = 6
HW2 = H2 * WP2
R2 = _rup(OFF2 + (H2 + 4) * WP2 + 4, 8)

K_FC1 = 64 * 7 * 7
K_PAD = 3200


def _c1_body(x_ref, w_ref, b_ref, o_ref, acc_ref):
    chunk = 128
    for c in range(HW1 // chunk):
        base = c * chunk
        acc = jnp.zeros((chunk, 128), jnp.float32)
        for k in range(25):
            dy, dx = k // 5, k % 5
            xs = x_ref[0, pl.ds(base + dy * WP1 + dx, chunk), :]
            acc = acc + xs * w_ref[k]
        acc_ref[pl.ds(base, chunk), :] = jnp.maximum(acc + b_ref[...], 0.0)

    o_ref[...] = jnp.zeros_like(o_ref)
    for ho in range(H1 // 2):
        p = None
        for r in (0, 1):
            for s in (0, 1):
                v = acc_ref[pl.ds((2 * ho + r) * WP1 + s, W1 // 2, stride=2), :]
                p = v if p is None else jnp.maximum(p, v)
        row = OFF2 + (ho + 2) * WP2 + 2
        for i in range(4):
            o_ref[i, pl.ds(row, W1 // 2), :] = p[:, i * 32:(i + 1) * 32]


def _c1_call(x_rep, w, b):
    g = x_rep.shape[0]
    return pl.pallas_call(
        _c1_body,
        out_shape=jax.ShapeDtypeStruct((4 * g, R2, 32), jnp.float32),
        grid=(g,),
        in_specs=[
            pl.BlockSpec((1, R1, 128), lambda i: (i, 0, 0)),
            pl.BlockSpec((25, 1, 128), lambda i: (0, 0, 0)),
            pl.BlockSpec((1, 128), lambda i: (0, 0)),
        ],
        out_specs=pl.BlockSpec((4, R2, 32), lambda i: (i, 0, 0)),
        scratch_shapes=[pltpu.VMEM((HW1, 128), jnp.float32)],
        compiler_params=pltpu.CompilerParams(dimension_semantics=("parallel",)),
    )(x_rep, w, b)


def _c2_body(x_ref, w_ref, b_ref, o_ref, acc_ref, *, nb):
    chunk = 48
    for img in range(nb):
        for c in range(HW2 // chunk):
            base = c * chunk
            acc = jnp.zeros((chunk, 64), jnp.float32)
            for dy in range(5):
                xs = jnp.concatenate(
                    [x_ref[img, pl.ds(OFF2 + base + dy * WP2 + dx, chunk), :]
                     for dx in range(5)], axis=-1)
                acc = acc + jnp.dot(xs.astype(w_ref.dtype), w_ref[dy],
                                    preferred_element_type=jnp.float32)
            acc_ref[pl.ds(base, chunk), :] = jnp.maximum(acc + b_ref[...], 0.0)

        for ho in range(H2 // 2):
            p = None
            for r in (0, 1):
                for s in (0, 1):
                    v = acc_ref[pl.ds((2 * ho + r) * WP2 + s, W2 // 2, stride=2), :]
                    p = v if p is None else jnp.maximum(p, v)
            o_ref[img, ho, :, :] = p.astype(o_ref.dtype)


def _c2_call(x, w, b, *, nb):
    n8 = x.shape[0]
    body = functools.partial(_c2_body, nb=nb)
    return pl.pallas_call(
        body,
        out_shape=jax.ShapeDtypeStruct((n8, 7, 7, 64), jnp.bfloat16),
        grid=(n8 // nb,),
        in_specs=[
            pl.BlockSpec((nb, R2, 32), lambda i: (i, 0, 0)),
            pl.BlockSpec((5, 160, 64), lambda i: (0, 0, 0)),
            pl.BlockSpec((1, 64), lambda i: (0, 0)),
        ],
        out_specs=pl.BlockSpec((nb, 7, 7, 64), lambda i: (i, 0, 0, 0)),
        scratch_shapes=[pltpu.VMEM((HW2, 64), jnp.float32)],
        compiler_params=pltpu.CompilerParams(dimension_semantics=("parallel",)),
    )(x, w, b)


def _head_body(x_ref, w1_ref, b1_ref, w2_ref, b2_ref, o_ref):
    h = jnp.dot(x_ref[...], w1_ref[...], preferred_element_type=jnp.float32)
    h = jnp.maximum(h + b1_ref[...], 0.0)
    logits = jnp.dot(h.astype(w2_ref.dtype), w2_ref[...],
                     preferred_element_type=jnp.float32) + b2_ref[...]
    m = jnp.max(logits, axis=-1, keepdims=True)
    s = logits - m
    lse = jnp.log(jnp.sum(jnp.exp(s), axis=-1, keepdims=True))
    o_ref[...] = s - lse


def _head_call(x, w1, b1, w2, b2):
    n8, kp = x.shape
    mt = 256
    return pl.pallas_call(
        _head_body,
        out_shape=jax.ShapeDtypeStruct((n8, 128), jnp.float32),
        grid=(n8 // mt,),
        in_specs=[
            pl.BlockSpec((mt, K_PAD), lambda i: (i, 0)),
            pl.BlockSpec((K_PAD, 1024), lambda i: (0, 0)),
            pl.BlockSpec((1, 1024), lambda i: (0, 0)),
            pl.BlockSpec((1024, 128), lambda i: (0, 0)),
            pl.BlockSpec((1, 128), lambda i: (0, 0)),
        ],
        out_specs=pl.BlockSpec((mt, 128), lambda i: (i, 0)),
        compiler_params=pltpu.CompilerParams(
            dimension_semantics=("parallel",)),
    )(x, w1, b1, w2, b2)


def kernel(x_nchw, conv1_w, conv1_b, conv2_w, conv2_b, fc1_w, fc1_b, fc2_w, fc2_b):
    N = x_nchw.shape[0]
    n8 = _rup(max(N, 1), 8)
    g = n8 // 4
    nb = 4 if n8 <= 8 else 8

    x = x_nchw.reshape(N, H1, W1)
    x = jnp.pad(x, ((0, n8 - N), (2, 2), (2, 2)))
    x = x.reshape(n8, (H1 + 4) * WP1)
    x = jnp.pad(x, ((0, 0), (0, R1 - (H1 + 4) * WP1)))
    x = x.reshape(g, 4, R1).transpose(0, 2, 1)
    x = jnp.broadcast_to(x[..., None], (g, R1, 4, 32)).reshape(g, R1, 128)

    a1 = _c1_call(x, conv1_w, conv1_b)
    a2 = _c2_call(a1, conv2_w, conv2_b, nb=nb)

    xf = a2.reshape(n8, K_FC1)
    xf = jnp.pad(xf, ((0, 0), (0, K_PAD - K_FC1)))
    out = _head_call(xf, fc1_w, fc1_b, fc2_w, fc2_b)
    return out[:N, :10]
